# Initial kernel scaffold; baseline (speedup 1.0000x reference)
import jax
import jax.numpy as jnp
from jax.experimental import pallas as pl


def kernel(X_write, X_read, slots, W0, ln_write_g, ln_write_b, W_V, W_K, slot_key_bias, W_gate_w, W_gate_b, ln_strand_g, ln_strand_b, Ws_Q, Ws_K, Ws_V, lambda_strand, W_Q_read, W_K_read, W_V_read, render_coeffs, slot_temps):
    raise NotImplementedError("write your pallas kernel here")


# speedup vs baseline: 5.8501x; 5.8501x over previous
import math
import functools

import jax
import jax.numpy as jnp
from jax.experimental import pallas as pl
from jax.experimental.pallas import tpu as pltpu

B, T, D = 2, 2048, 1024
K, D_B = 7, 128
TOPK, WINDOW = 16, 64
GATE_CAP, STRAND_CAP = 0.5, 0.5
PHI = 1.618033988749895
PSI = -0.6180339887498949

TT = 512
NT = T // TT
NSB = TT // WINDOW
NCOL = (T // WINDOW)
NEG = -float("inf")

_INTERPRET = False


def _ln(x, g, b, eps=1e-5):
    m = jnp.mean(x, axis=-1, keepdims=True)
    v = jnp.mean((x - m) ** 2, axis=-1, keepdims=True)
    return (x - m) * jax.lax.rsqrt(v + eps) * g + b


def _band_kernel(xw_ref, xr_ref, w0_ref, g_ref, b_ref,
                 band_ref, rmax_ref, qsum_ref, prev_ref):
    it = pl.program_id(1)

    @pl.when(it == 0)
    def _():
        prev_ref[...] = jnp.zeros_like(prev_ref)

    x = xw_ref[0]
    xn = _ln(x, g_ref[...], b_ref[...])
    xw0 = jax.lax.dot_general(xn, w0_ref[...], (((1,), (1,)), ((), ())),
                              preferred_element_type=jnp.float32)

    il = jax.lax.broadcasted_iota(jnp.int32, (WINDOW, 2 * WINDOW), 0)
    il64 = jax.lax.broadcasted_iota(jnp.int32, (WINDOW, WINDOW), 0)
    w64 = jax.lax.broadcasted_iota(jnp.int32, (WINDOW, WINDOW), 1)

    rmax_cols = []
    for k in range(NSB):
        xsub = xn[k * WINDOW:(k + 1) * WINDOW]
        if k == 0:
            ext = jnp.concatenate([prev_ref[...], xw0[:WINDOW]], axis=0)
        else:
            ext = xw0[(k - 1) * WINDOW:(k + 1) * WINDOW]
        h = jax.lax.dot_general(xsub, ext, (((1,), (1,)), ((), ())),
                                preferred_element_type=jnp.float32)
        for s in (32, 16, 8, 4, 2, 1):
            shifted = jnp.concatenate(
                [h[:, s:], jnp.zeros((WINDOW, s), jnp.float32)], axis=1)
            h = jnp.where((il & s) != 0, shifted, h)
        bsb = h[:, :WINDOW]
        i_g = it * TT + k * WINDOW + il64
        j_g = i_g - WINDOW + w64
        bsb = jnp.where((j_g >= 0) & (i_g >= 1), bsb, NEG)
        band_ref[0, k * WINDOW:(k + 1) * WINDOW, :] = bsb
        rmax_cols.append(jnp.max(bsb, axis=1, keepdims=True))

    rmax_ref[0, 0] = jnp.concatenate(rmax_cols, axis=1)

    s = jnp.sum(xr_ref[0], axis=0, keepdims=True)

    @pl.when(it == 0)
    def _():
        qsum_ref[0] = s

    @pl.when(it != 0)
    def _():
        qsum_ref[0] = qsum_ref[0] + s

    prev_ref[...] = xw0[TT - WINDOW:]


def _run_band(X_write, X_read, W0, g1, b1):
    return pl.pallas_call(
        _band_kernel,
        grid=(B, NT),
        in_specs=[
            pl.BlockSpec((1, TT, D), lambda b, i: (b, i, 0)),
            pl.BlockSpec((1, TT, D), lambda b, i: (b, i, 0)),
            pl.BlockSpec((D, D), lambda b, i: (0, 0)),
            pl.BlockSpec((1, D), lambda b, i: (0, 0)),
            pl.BlockSpec((1, D), lambda b, i: (0, 0)),
        ],
        out_specs=[
            pl.BlockSpec((1, TT, WINDOW), lambda b, i: (b, i, 0)),
            pl.BlockSpec((1, 1, WINDOW, NSB), lambda b, i: (b, i, 0, 0)),
            pl.BlockSpec((1, 1, D), lambda b, i: (b, 0, 0)),
        ],
        out_shape=[
            jax.ShapeDtypeStruct((B, T, WINDOW), jnp.float32),
            jax.ShapeDtypeStruct((B, NT, WINDOW, NSB), jnp.float32),
            jax.ShapeDtypeStruct((B, 1, D), jnp.float32),
        ],
        scratch_shapes=[pltpu.VMEM((WINDOW, D), jnp.float32)],
        compiler_params=pltpu.CompilerParams(
            dimension_semantics=("arbitrary", "arbitrary")),
        interpret=_INTERPRET,
    )(X_write, X_read, W0, g1, b1)


def _slots_kernel(band_ref, rmax_ref, xw_ref, qsum_ref, slots_ref,
                  w0_ref, lng_ref, lnb_ref, wv_ref, wk_ref, skb_ref,
                  wgw_ref, wgb_ref, sg_ref, sb_ref, wsq_ref, wsk_ref,
                  wsv_ref, lam_ref, wkr_ref, wvr_ref, rc_ref, st_ref,
                  slots_out, kr_out, vg_out, lp_out, c2_out):
    rmax = rmax_ref[0]
    itx = jax.lax.broadcasted_iota(jnp.int32, (NT, WINDOW, NSB), 0)
    ril = jax.lax.broadcasted_iota(jnp.int32, (NT, WINDOW, NSB), 1)
    rsb = jax.lax.broadcasted_iota(jnp.int32, (NT, WINDOW, NSB), 2)
    rid = (itx * NSB + rsb) * WINDOW + ril

    rrows = []
    rids = []
    for _ in range(TOPK):
        m = jnp.max(rmax)
        r = jnp.min(jnp.where(rmax == m, rid, jnp.int32(2 ** 30)))
        rids.append(r)
        rrows.append(band_ref[0, pl.ds(r, 1), :])
        rmax = jnp.where(rid == r, NEG, rmax)

    cand = jnp.concatenate(rrows, axis=0)
    ci_rows = jnp.concatenate(
        [jnp.full((1, WINDOW), r, jnp.int32) for r in rids], axis=0)
    wio = jax.lax.broadcasted_iota(jnp.int32, (TOPK, WINDOW), 1)
    glin = ci_rows * WINDOW + wio

    lane16 = jax.lax.broadcasted_iota(jnp.int32, (1, TOPK), 1)
    vals = jnp.zeros((1, TOPK), jnp.float32)
    i_list, j_list = [], []
    for kk in range(TOPK):
        m = jnp.max(cand)
        g = jnp.min(jnp.where(cand == m, glin, jnp.int32(2 ** 30)))
        i_k = g // WINDOW
        j_k = i_k - WINDOW + (g % WINDOW)
        i_list.append(i_k)
        j_list.append(j_k)
        vals = jnp.where(lane16 == kk, m, vals)
        cand = jnp.where(glin == g, NEG, cand)

    newer = jnp.concatenate([xw_ref[0, pl.ds(i, 1), :] for i in i_list], axis=0)
    older = jnp.concatenate([xw_ref[0, pl.ds(j, 1), :] for j in j_list], axis=0)
    lng, lnb = lng_ref[...], lnb_ref[...]
    xn_new = _ln(newer, lng, lnb)
    xn_old = _ln(older, lng, lnb)

    alpha = jax.nn.softmax(vals, axis=-1)
    rel = jax.lax.dot_general(xn_new - xn_old, wv_ref[...],
                              (((1,), (1,)), ((), ())),
                              preferred_element_type=jnp.float32)
    r_t = jax.lax.dot_general(alpha, rel, (((1,), (0,)), ((), ())),
                              preferred_element_type=jnp.float32)

    slots = slots_ref[0]
    slot_keys = jax.lax.dot_general(slots, wk_ref[...], (((1,), (1,)), ((), ())),
                                    preferred_element_type=jnp.float32)
    slot_keys = slot_keys + skb_ref[...]
    compat = jnp.sum(slot_keys * r_t, axis=1, keepdims=True) / math.sqrt(D_B)
    compat = compat * jax.nn.softplus(st_ref[...])
    cmax = jnp.max(compat, axis=0, keepdims=True)
    ce = jnp.exp(compat - cmax)
    slot_w = ce / jnp.sum(ce, axis=0, keepdims=True)
    weighted_r = slot_w * r_t

    gate_in = jnp.concatenate([slots, weighted_r], axis=1)
    g = jax.nn.sigmoid(
        jax.lax.dot_general(gate_in, wgw_ref[...], (((1,), (1,)), ((), ())),
                            preferred_element_type=jnp.float32) + wgb_ref[...])
    slots_updated = (1.0 - g) * slots + g * weighted_r

    b_ln = _ln(slots_updated, sg_ref[...], sb_ref[...])
    sq = jax.lax.dot_general(b_ln, wsq_ref[...], (((1,), (1,)), ((), ())),
                             preferred_element_type=jnp.float32)
    sk = jax.lax.dot_general(b_ln, wsk_ref[...], (((1,), (1,)), ((), ())),
                             preferred_element_type=jnp.float32)
    sv = jax.lax.dot_general(b_ln, wsv_ref[...], (((1,), (1,)), ((), ())),
                             preferred_element_type=jnp.float32)
    sa = jax.lax.dot_general(sq, sk, (((1,), (1,)), ((), ())),
                             preferred_element_type=jnp.float32) / math.sqrt(D_B)
    sa = jax.nn.softmax(sa, axis=-1)
    ctx = jax.lax.dot_general(sa, sv, (((1,), (0,)), ((), ())),
                              preferred_element_type=jnp.float32)
    lam = jnp.clip(jnp.tanh(lam_ref[...]), -STRAND_CAP, STRAND_CAP)
    slots_next = slots_updated + ctx * lam
    slots_out[0] = slots_next

    lp_out[0] = jnp.sum((slots_next - slots) ** 2, axis=(0, 1),
                        keepdims=True) / (K * D_B)

    nn_ = jnp.sqrt(jnp.sum(slots_next ** 2, axis=1, keepdims=True))
    bn = slots_next / jnp.maximum(nn_, 1e-12)
    cos = jax.lax.dot_general(bn, bn, (((1,), (1,)), ((), ())),
                              preferred_element_type=jnp.float32)
    r7 = jax.lax.broadcasted_iota(jnp.int32, (K, K), 0)
    c7 = jax.lax.broadcasted_iota(jnp.int32, (K, K), 1)
    c2_out[0] = jnp.sum(jnp.where(r7 == c7, 0.0, cos * cos), axis=(0, 1),
                        keepdims=True) / (K * (K - 1))

    k_r = jax.lax.dot_general(slots_next, wkr_ref[...], (((1,), (1,)), ((), ())),
                              preferred_element_type=jnp.float32)
    v_r = jax.lax.dot_general(slots_next, wvr_ref[...], (((1,), (1,)), ((), ())),
                              preferred_element_type=jnp.float32)
    q_mean = qsum_ref[0] / T
    u = jax.lax.dot_general(q_mean, w0_ref[...], (((1,), (1,)), ((), ())),
                            preferred_element_type=jnp.float32)
    denom = PHI - PSI
    u_phi = (u - PSI * q_mean) / denom
    u_psi = (PHI * q_mean - u) / denom
    rc = rc_ref[...]
    gate = jnp.tanh(rc[0:K] + rc[K:2 * K] * u_phi + rc[2 * K:3 * K] * u_psi)
    gate = jnp.clip(gate, -GATE_CAP, GATE_CAP)
    v_g = v_r * gate
    z1 = jnp.zeros((1, D_B), jnp.float32)
    kr_out[0] = jnp.concatenate([k_r, z1], axis=0)
    vg_out[0] = jnp.concatenate([v_g, jnp.zeros((1, D), jnp.float32)], axis=0)


def _run_slots(band, rmax, X_write, qsum, slots, W0, lng, lnb, W_V, W_K,
               skb, Wgw, wgb, sg, sb, WsQ, WsK, WsV, lam, WKr, WVr, rc, st):
    full = lambda *shape: pl.BlockSpec(shape, lambda b: tuple(0 for _ in shape))
    return pl.pallas_call(
        _slots_kernel,
        grid=(B,),
        in_specs=[
            pl.BlockSpec((1, T, WINDOW), lambda b: (b, 0, 0)),
            pl.BlockSpec((1, NT, WINDOW, NSB), lambda b: (b, 0, 0, 0)),
            pl.BlockSpec((1, T, D), lambda b: (b, 0, 0)),
            pl.BlockSpec((1, 1, D), lambda b: (b, 0, 0)),
            pl.BlockSpec((1, K, D_B), lambda b: (b, 0, 0)),
            full(D, D), full(1, D), full(1, D), full(D_B, D), full(D_B, D_B),
            full(K, D_B), full(D_B, 2 * D_B), full(1, D_B), full(1, D_B),
            full(1, D_B), full(D_B, D_B), full(D_B, D_B), full(D_B, D_B),
            full(1, D_B), full(D_B, D_B), full(D, D_B), full(3 * K, D),
            full(K, 1),
        ],
        out_specs=[
            pl.BlockSpec((1, K, D_B), lambda b: (b, 0, 0)),
            pl.BlockSpec((1, 8, D_B), lambda b: (b, 0, 0)),
            pl.BlockSpec((1, 8, D), lambda b: (b, 0, 0)),
            pl.BlockSpec((1, 1, 1), lambda b: (b, 0, 0)),
            pl.BlockSpec((1, 1, 1), lambda b: (b, 0, 0)),
        ],
        out_shape=[
            jax.ShapeDtypeStruct((B, K, D_B), jnp.float32),
            jax.ShapeDtypeStruct((B, 8, D_B), jnp.float32),
            jax.ShapeDtypeStruct((B, 8, D), jnp.float32),
            jax.ShapeDtypeStruct((B, 1, 1), jnp.float32),
            jax.ShapeDtypeStruct((B, 1, 1), jnp.float32),
        ],
        compiler_params=pltpu.CompilerParams(
            dimension_semantics=("arbitrary",)),
        interpret=_INTERPRET,
    )(band, rmax, X_write, qsum, slots, W0, lng, lnb, W_V, W_K, skb, Wgw,
      wgb, sg, sb, WsQ, WsK, WsV, lam, WKr, WVr, rc, st)


def _read_kernel(xr_ref, wq_ref, kr_ref, vg_ref, z_ref):
    q = jax.lax.dot_general(xr_ref[0], wq_ref[...], (((1,), (1,)), ((), ())),
                            preferred_element_type=jnp.float32)
    scores = jax.lax.dot_general(q, kr_ref[0], (((1,), (1,)), ((), ())),
                                 preferred_element_type=jnp.float32)
    scores = scores / math.sqrt(D_B)
    lane8 = jax.lax.broadcasted_iota(jnp.int32, (TT, 8), 1)
    scores = jnp.where(lane8 < K, scores, NEG)
    attn = jax.nn.softmax(scores, axis=-1)
    z_ref[0] = jax.lax.dot_general(attn, vg_ref[0], (((1,), (0,)), ((), ())),
                                   preferred_element_type=jnp.float32)


def _run_read(X_read, WQr, kr_pad, vg_pad):
    return pl.pallas_call(
        _read_kernel,
        grid=(B, NT),
        in_specs=[
            pl.BlockSpec((1, TT, D), lambda b, i: (b, i, 0)),
            pl.BlockSpec((D_B, D), lambda b, i: (0, 0)),
            pl.BlockSpec((1, 8, D_B), lambda b, i: (b, 0, 0)),
            pl.BlockSpec((1, 8, D), lambda b, i: (b, 0, 0)),
        ],
        out_specs=pl.BlockSpec((1, TT, D), lambda b, i: (b, i, 0)),
        out_shape=jax.ShapeDtypeStruct((B, T, D), jnp.float32),
        compiler_params=pltpu.CompilerParams(
            dimension_semantics=("arbitrary", "arbitrary")),
        interpret=_INTERPRET,
    )(X_read, WQr, kr_pad, vg_pad)


@jax.jit
def kernel(X_write, X_read, slots, W0, ln_write_g, ln_write_b, W_V, W_K,
           slot_key_bias, W_gate_w, W_gate_b, ln_strand_g, ln_strand_b,
           Ws_Q, Ws_K, Ws_V, lambda_strand, W_Q_read, W_K_read, W_V_read,
           render_coeffs, slot_temps):
    g1 = ln_write_g.reshape(1, D)
    b1 = ln_write_b.reshape(1, D)
    band, rmax, qsum = _run_band(X_write, X_read, W0, g1, b1)

    rc = jnp.concatenate(
        [render_coeffs[:, 0, :], render_coeffs[:, 1, :], render_coeffs[:, 2, :]],
        axis=0)
    slots_next, kr_pad, vg_pad, lp, c2 = _run_slots(
        band, rmax, X_write, qsum, slots, W0, g1, b1, W_V, W_K,
        slot_key_bias, W_gate_w, W_gate_b.reshape(1, D_B),
        ln_strand_g.reshape(1, D_B), ln_strand_b.reshape(1, D_B),
        Ws_Q, Ws_K, Ws_V, lambda_strand.reshape(1, D_B),
        W_K_read, W_V_read, rc, slot_temps.reshape(K, 1))

    z = _run_read(X_read, W_Q_read, kr_pad, vg_pad)

    l_persist = jnp.mean(lp)
    l_div = jnp.mean(c2)
    return z, slots_next, l_persist, l_div
